# Initial kernel scaffold; baseline (speedup 1.0000x reference)
import jax
import jax.numpy as jnp
from jax.experimental import pallas as pl


def kernel(token_embedding, pos_table):
    raise NotImplementedError("write your pallas kernel here")


# speedup vs baseline: 1.3723x; 1.3723x over previous
import jax
import jax.numpy as jnp
from jax.experimental import pallas as pl


_SEQ_BLOCK = 512


def _add_body(tok_ref, pos_ref, out_ref):
    out_ref[...] = tok_ref[...] + pos_ref[...]


def kernel(token_embedding, pos_table):
    B, S, E = token_embedding.shape
    pos = jax.lax.slice(pos_table, (0, 0), (S, E))
    bs = _SEQ_BLOCK
    grid = (S // bs, B)
    return pl.pallas_call(
        _add_body,
        grid=grid,
        in_specs=[
            pl.BlockSpec((1, bs, E), lambda i, j: (j, i, 0)),
            pl.BlockSpec((bs, E), lambda i, j: (i, 0)),
        ],
        out_specs=pl.BlockSpec((1, bs, E), lambda i, j: (j, i, 0)),
        out_shape=jax.ShapeDtypeStruct((B, S, E), token_embedding.dtype),
    )(token_embedding, pos)
